# baseline (speedup 1.0000x reference)
import functools
import math

import jax
import jax.numpy as jnp
from jax import lax
from jax.experimental import pallas as pl
from jax.experimental.pallas import tpu as pltpu
from jax.experimental.pallas import tpu_sc as plsc

D_MODEL = 64
SCALE = math.sqrt(D_MODEL)
NUM_CORES = 2
NUM_SUBCORES = 16
NW = NUM_CORES * NUM_SUBCORES
BLOCK = 128
CHUNK = 64
NBUF = 4
L = 16


@functools.lru_cache(maxsize=None)
def _make_embed(nbatch: int, seq: int, vocab2: int):
    assert nbatch == NW * BLOCK and (2 * seq) % NBUF == 0
    n_groups = 2 * seq // NBUF
    mesh = plsc.VectorSubcoreMesh(core_axis_name="c", subcore_axis_name="s")

    @functools.partial(
        pl.kernel,
        mesh=mesh,
        compiler_params=pltpu.CompilerParams(use_tc_tiling_on_sc=True),
        out_type=jax.ShapeDtypeStruct((seq, nbatch, D_MODEL), jnp.float32),
        scratch_types=[
            pltpu.VMEM((seq, BLOCK), jnp.int32),
            pltpu.VMEM((NBUF, CHUNK), jnp.int32),
            pltpu.VMEM((NBUF, CHUNK, 2 * D_MODEL), jnp.float32),
            pltpu.VMEM((NBUF, CHUNK, D_MODEL), jnp.float32),
            pltpu.SemaphoreType.DMA((NBUF,)),
            pltpu.SemaphoreType.DMA((NBUF,)),
        ],
    )
    def embed(xt_hbm, lut_hbm, out_hbm, idx_v, half_v, gbuf, wbuf, gsem, wsem):
        wid = lax.axis_index("s") * NUM_CORES + lax.axis_index("c")
        b0 = wid * BLOCK
        pltpu.sync_copy(xt_hbm.at[:, pl.ds(b0, BLOCK)], idx_v)

        def start_gather(s, h, b):
            for m in range(CHUNK // L):
                sl = pl.ds(m * L, L)
                half_v[b, sl] = lax.shift_right_logical(
                    idx_v[s, pl.ds(h * CHUNK + m * L, L)], 1
                )
            pltpu.async_copy(lut_hbm.at[half_v.at[b]], gbuf.at[b], gsem.at[b])

        for b in range(NBUF):
            start_gather(b // 2, b % 2, b)

        def group_body(cc, carry):
            for k in range(NBUF):
                h = k % 2
                s = cc * (NBUF // 2) + k // 2
                pltpu.make_async_copy(
                    lut_hbm.at[half_v.at[0]], gbuf.at[k], gsem.at[k]
                ).wait()

                @pl.when(cc > 0)
                def _wait_wb():
                    pltpu.make_async_copy(
                        wbuf.at[k], out_hbm.at[0, pl.ds(0, CHUNK)], wsem.at[k]
                    ).wait()

                def grp_body(m, carry2):
                    pv = idx_v[s, pl.ds(h * CHUNK + m * L, L)]
                    for ri in range(L):
                        r = m * L + ri
                        off = (pv[ri] & 1) * D_MODEL
                        for j in range(D_MODEL // L):
                            v = gbuf[k, r, pl.ds(off + j * L, L)]
                            wbuf[k, r, pl.ds(j * L, L)] = v * SCALE
                    return carry2

                lax.fori_loop(0, CHUNK // L, grp_body, 0)

                pltpu.async_copy(
                    wbuf.at[k],
                    out_hbm.at[s, pl.ds(b0 + h * CHUNK, CHUNK)],
                    wsem.at[k],
                )

                @pl.when(cc < n_groups - 1)
                def _next_gather():
                    start_gather(s + NBUF // 2, h, k)

            return carry

        lax.fori_loop(0, n_groups, group_body, 0)

        for b in range(NBUF):
            pltpu.make_async_copy(
                wbuf.at[b], out_hbm.at[0, pl.ds(0, CHUNK)], wsem.at[b]
            ).wait()

    return embed


def kernel(x, lut):
    nb, seq = x.shape
    xt = jnp.transpose(x).astype(jnp.int32)
    lut2 = lut.reshape(lut.shape[0] // 2, 2 * D_MODEL)
    out_t = _make_embed(nb, seq, lut2.shape[0])(xt, lut2)
    return jnp.transpose(out_t, (1, 0, 2))
